# Initial kernel scaffold; baseline (speedup 1.0000x reference)
import jax
import jax.numpy as jnp
from jax.experimental import pallas as pl


def kernel(user_user_id, user_age, user_gender, user_time_of_day, user_day_of_week, user_liked_tags, user_disliked_tags, user_allergy_tags, item_dish_id, item_store_id, item_category, item_tags, item_price, item_rating, item_time_of_day, item_day_of_week, user_embedding, user_age_W, user_age_b, user_gender_emb, user_time_W, user_time_b, user_day_emb, dish_embedding, store_embedding, category_embedding, dish_price_W, dish_price_b, dish_rating_W, dish_rating_b, dish_time_W, dish_time_b, dish_day_emb, tag_embedding, user_proj_W, user_proj_b, item_proj_W, item_proj_b):
    raise NotImplementedError("write your pallas kernel here")


# speedup vs baseline: 15.2988x; 15.2988x over previous
import functools

import jax
import jax.numpy as jnp
from jax import lax
from jax.experimental import pallas as pl
from jax.experimental.pallas import tpu as pltpu
from jax.experimental.pallas import tpu_sc as plsc

B = 16384
D = 64
T = 20
NC, NS = 2, 16
NW = NC * NS
BC = B // NW
CH = 128
NCHUNK = BC // CH


def _sc_body(tag_t, user_t, dish_t, store_t, cat_t,
             liked_T, disliked_T, allergy_T, tags_T,
             uid, did, sid, cid,
             acc_l, acc_d, acc_a, acc_t, u_rows, d_rows, s_rows, c_rows,
             idx_v, rows_v, pout_v, gidx_v, g64_v, g32_v, g16_v, sem):
    wid = lax.axis_index("s") * NC + lax.axis_index("c")
    base = wid * BC

    def plain(table, ids_hbm, out_hbm, buf):
        def body(c, carry):
            off = pl.multiple_of(base + c * CH, CH)
            pltpu.sync_copy(ids_hbm.at[pl.ds(off, CH)], gidx_v)
            pltpu.async_copy(table.at[gidx_v], buf, sem).wait()
            pltpu.sync_copy(buf, out_hbm.at[pl.ds(off, CH)])
            return carry
        lax.fori_loop(0, NCHUNK, body, 0)

    plain(user_t, uid, u_rows, g64_v)
    plain(dish_t, did, d_rows, g64_v)
    plain(store_t, sid, s_rows, g32_v)
    plain(cat_t, cid, c_rows, g16_v)

    def pool(idxT, out_hbm):
        def body(c, carry):
            off = pl.multiple_of(base + c * CH, CH)
            pltpu.sync_copy(idxT.at[:, pl.ds(off, CH)], idx_v)
            copies = [
                pltpu.async_copy(tag_t.at[idx_v.at[k]], rows_v.at[k], sem)
                for k in range(T)
            ]
            for cp in copies:
                cp.wait()

            def red(i, rcarry):
                for h in range(2):
                    sl = pl.ds(h * 16, 16)
                    s = rows_v[0, i, sl]
                    for k in range(1, T):
                        s = s + rows_v[k, i, sl]
                    pout_v[i, sl] = s
                return rcarry
            lax.fori_loop(0, CH, red, 0)
            pltpu.sync_copy(pout_v, out_hbm.at[pl.ds(off, CH)])
            return carry
        lax.fori_loop(0, NCHUNK, body, 0)

    pool(liked_T, acc_l)
    pool(disliked_T, acc_d)
    pool(allergy_T, acc_a)
    pool(tags_T, acc_t)


@functools.cache
def _sc_gather_call():
    return pl.kernel(
        _sc_body,
        out_type=(
            jax.ShapeDtypeStruct((B, 32), jnp.float32),
            jax.ShapeDtypeStruct((B, 32), jnp.float32),
            jax.ShapeDtypeStruct((B, 32), jnp.float32),
            jax.ShapeDtypeStruct((B, 32), jnp.float32),
            jax.ShapeDtypeStruct((B, 64), jnp.float32),
            jax.ShapeDtypeStruct((B, 64), jnp.float32),
            jax.ShapeDtypeStruct((B, 32), jnp.float32),
            jax.ShapeDtypeStruct((B, 16), jnp.float32),
        ),
        mesh=plsc.VectorSubcoreMesh(core_axis_name="c", subcore_axis_name="s",
                                    num_cores=NC, num_subcores=NS),
        scratch_types=[
            pltpu.VMEM((T, CH), jnp.int32),
            pltpu.VMEM((T, CH, 32), jnp.float32),
            pltpu.VMEM((CH, 32), jnp.float32),
            pltpu.VMEM((CH,), jnp.int32),
            pltpu.VMEM((CH, 64), jnp.float32),
            pltpu.VMEM((CH, 32), jnp.float32),
            pltpu.VMEM((CH, 16), jnp.float32),
            pltpu.SemaphoreType.DMA,
        ],
        compiler_params=pltpu.CompilerParams(use_tc_tiling_on_sc=False),
    )


BN = 2048


def _tc_body(accl, accd, acca, acct, urow, drow, srow, crow,
             il, idd, ia, it, uscal, iscal, ids,
             Wu, bu, Wi, bi, gemb, uday, iday,
             ageW, ageb, utW, utb, prW, prb, rtW, rtb, itW, itb, row0,
             uo, io, doto):
    f32 = jnp.float32
    dot = functools.partial(lax.dot, preferred_element_type=f32)
    r0 = row0[...]

    def pool(acc_ref, idx_ref):
        cnt = jnp.sum((idx_ref[...] != 0).astype(f32), axis=1, keepdims=True)
        return (acc_ref[...] - (float(T) - cnt) * r0) / (cnt + 1e-8)

    liked = pool(accl, il)
    disl = pool(accd, idd)
    alle = pool(acca, ia)
    tagv = pool(acct, it)

    def onehot(col, n):
        return (lax.broadcasted_iota(jnp.int32, (BN, n), 1) == col).astype(f32)

    Wu_ = Wu[...]
    age = uscal[:, 0:1]
    utod = uscal[:, 1:2]
    u = dot(urow[...], Wu_[0:64])
    u += dot(liked, Wu_[112:144])
    u += dot(disl, Wu_[144:176])
    u += dot(alle, Wu_[176:208])
    u += age * dot(ageW[...], Wu_[64:80])
    u += utod * dot(utW[...], Wu_[96:104])
    u += dot(onehot(ids[:, 0:1], 3), dot(gemb[...], Wu_[80:96]))
    u += dot(onehot(ids[:, 1:2], 7), dot(uday[...], Wu_[104:112]))
    u += bu[...] + dot(ageb[...], Wu_[64:80]) + dot(utb[...], Wu_[96:104])
    nu = jnp.sqrt(jnp.sum(u * u, axis=1, keepdims=True))
    un = u / jnp.maximum(nu, 1e-12)

    Wi_ = Wi[...]
    price = iscal[:, 0:1]
    rating = iscal[:, 1:2]
    itod = iscal[:, 2:3]
    iv = dot(drow[...], Wi_[0:64])
    iv += dot(srow[...], Wi_[64:96])
    iv += dot(tagv, Wi_[96:128])
    iv += dot(crow[...], Wi_[128:144])
    iv += price * dot(prW[...], Wi_[144:160])
    iv += rating * dot(rtW[...], Wi_[160:168])
    iv += itod * dot(itW[...], Wi_[168:176])
    iv += dot(onehot(ids[:, 2:3], 7), dot(iday[...], Wi_[176:184]))
    iv += (bi[...] + dot(prb[...], Wi_[144:160]) + dot(rtb[...], Wi_[160:168])
           + dot(itb[...], Wi_[168:176]))
    ni = jnp.sqrt(jnp.sum(iv * iv, axis=1, keepdims=True))
    ivn = iv / jnp.maximum(ni, 1e-12)

    uo[...] = un
    io[...] = ivn
    doto[...] = jnp.sum(un * ivn, axis=1, keepdims=True)


def _row_spec(k):
    return pl.BlockSpec((BN, k), lambda i: (i, 0))


def _full_spec(shape):
    return pl.BlockSpec(shape, lambda i: (0,) * len(shape))


def kernel(user_user_id, user_age, user_gender, user_time_of_day,
           user_day_of_week, user_liked_tags, user_disliked_tags,
           user_allergy_tags, item_dish_id, item_store_id, item_category,
           item_tags, item_price, item_rating, item_time_of_day,
           item_day_of_week, user_embedding, user_age_W, user_age_b,
           user_gender_emb, user_time_W, user_time_b, user_day_emb,
           dish_embedding, store_embedding, category_embedding,
           dish_price_W, dish_price_b, dish_rating_W, dish_rating_b,
           dish_time_W, dish_time_b, dish_day_emb, tag_embedding,
           user_proj_W, user_proj_b, item_proj_W, item_proj_b):
    i32 = jnp.int32
    f32 = jnp.float32
    liked_T = user_liked_tags.astype(i32).T
    disliked_T = user_disliked_tags.astype(i32).T
    allergy_T = user_allergy_tags.astype(i32).T
    tags_T = item_tags.astype(i32).T

    (accl, accd, acca, acct, urow, drow, srow, crow) = _sc_gather_call()(
        tag_embedding, user_embedding, dish_embedding, store_embedding,
        category_embedding, liked_T, disliked_T, allergy_T, tags_T,
        user_user_id.astype(i32), item_dish_id.astype(i32),
        item_store_id.astype(i32), item_category.astype(i32))

    uscal = jnp.stack([user_age, user_time_of_day], axis=1).astype(f32)
    iscal = jnp.stack([item_price, item_rating, item_time_of_day],
                      axis=1).astype(f32)
    ids = jnp.stack([user_gender, user_day_of_week, item_day_of_week],
                    axis=1).astype(i32)
    row0 = tag_embedding[0:1]

    u_in, i_in = 208, 184
    weights = dict(
        Wu=(user_proj_W, (u_in, D)), bu=(user_proj_b.reshape(1, D), (1, D)),
        Wi=(item_proj_W, (i_in, D)), bi=(item_proj_b.reshape(1, D), (1, D)),
        gemb=(user_gender_emb, (3, 16)), uday=(user_day_emb, (7, 8)),
        iday=(dish_day_emb, (7, 8)),
        ageW=(user_age_W, (1, 16)), ageb=(user_age_b.reshape(1, 16), (1, 16)),
        utW=(user_time_W, (1, 8)), utb=(user_time_b.reshape(1, 8), (1, 8)),
        prW=(dish_price_W, (1, 16)), prb=(dish_price_b.reshape(1, 16), (1, 16)),
        rtW=(dish_rating_W, (1, 8)), rtb=(dish_rating_b.reshape(1, 8), (1, 8)),
        itW=(dish_time_W, (1, 8)), itb=(dish_time_b.reshape(1, 8), (1, 8)),
        row0=(row0, (1, 32)),
    )

    in_specs = (
        [_row_spec(32)] * 4
        + [_row_spec(64), _row_spec(64), _row_spec(32), _row_spec(16)]
        + [_row_spec(T)] * 4
        + [_row_spec(2), _row_spec(3), _row_spec(3)]
        + [_full_spec(s) for (_, s) in weights.values()]
    )

    un, ivn, dotv = pl.pallas_call(
        _tc_body,
        grid=(B // BN,),
        in_specs=in_specs,
        out_specs=[_row_spec(D), _row_spec(D), _row_spec(1)],
        out_shape=[
            jax.ShapeDtypeStruct((B, D), f32),
            jax.ShapeDtypeStruct((B, D), f32),
            jax.ShapeDtypeStruct((B, 1), f32),
        ],
    )(accl, accd, acca, acct, urow, drow, srow, crow,
      user_liked_tags.astype(i32), user_disliked_tags.astype(i32),
      user_allergy_tags.astype(i32), item_tags.astype(i32),
      uscal, iscal, ids, *[w for (w, _) in weights.values()])

    return un, ivn, dotv.reshape(B)
